# baseline (speedup 1.0000x reference)
import functools

import jax
import jax.numpy as jnp
from jax import lax
from jax.experimental import pallas as pl
from jax.experimental.pallas import tpu as pltpu
from jax.experimental.pallas import tpu_sc as plsc

_D = 300
_DP = 384
_L = 50
_LP = 56
_LANES = 16
_GR = 2
_NBUF = 2
_OC = 16

_OFFS = tuple(list(range(0, 288, _LANES)) + [_D - _LANES])


def _build_sc_pool(btot):
    info = plsc.get_sparse_core_info()
    nc, ns = info.num_cores, info.num_subcores
    nw = nc * ns
    rpw = btot // nw
    assert btot % nw == 0 and rpw % _OC == 0
    nch = rpw // _OC
    mesh = plsc.VectorSubcoreMesh(core_axis_name="c", subcore_axis_name="s")

    @functools.partial(
        pl.kernel,
        out_type=jax.ShapeDtypeStruct((btot, _D), jnp.float32),
        mesh=mesh,
        scratch_types=[
            pltpu.VMEM((rpw * _LP,), jnp.int32),
            pltpu.VMEM((_NBUF, _GR * _LP, _DP), jnp.float32),
            pltpu.VMEM((2, _OC, _D), jnp.float32),
            pltpu.SemaphoreType.DMA((_NBUF,)),
            pltpu.SemaphoreType.DMA((2,)),
        ],
    )
    def sc_pool(x_hbm, emb_hbm, out_hbm, idx_v, ring_v, outb_v, gsem, osem):
        wid = lax.axis_index("s") * nc + lax.axis_index("c")
        base = wid * rpw
        pltpu.sync_copy(x_hbm.at[pl.ds(base * _LP, rpw * _LP)], idx_v)

        for g in range(_NBUF):
            pltpu.async_copy(
                emb_hbm.at[idx_v.at[pl.ds(g * _GR * _LP, _GR * _LP)]],
                ring_v.at[g], gsem.at[g])

        @pl.loop(0, nch)
        def _chunk(c):
            parity = lax.rem(c, 2)

            @pl.when(c >= 2)
            def _():
                pltpu.make_async_copy(
                    outb_v.at[parity], out_hbm.at[pl.ds(base, _OC)],
                    osem.at[parity]).wait()

            for j in range(0, _OC, _GR):
                row = c * _OC + j
                g = j // _GR
                slot = g % _NBUF
                pltpu.make_async_copy(
                    emb_hbm.at[idx_v.at[pl.ds(row * _LP, _GR * _LP)]],
                    ring_v.at[slot], gsem.at[slot]).wait()
                rv = ring_v.at[slot]
                for s in range(_GR):
                    acc0 = tuple(
                        rv[s * _LP, pl.ds(o, _LANES)] for o in _OFFS)

                    def _body(r, acc, rv=rv, s=s):
                        return tuple(
                            a + rv[s * _LP + r, pl.ds(o, _LANES)]
                            for a, o in zip(acc, _OFFS))

                    acc = lax.fori_loop(1, _L, _body, acc0)
                    for t, o in enumerate(_OFFS):
                        outb_v[parity, j + s, pl.ds(o, _LANES)] = acc[t]
                nxt = jnp.minimum(row + _NBUF * _GR, rpw - _GR)
                pltpu.async_copy(
                    emb_hbm.at[idx_v.at[pl.ds(nxt * _LP, _GR * _LP)]],
                    ring_v.at[slot], gsem.at[slot])

            pltpu.async_copy(
                outb_v.at[parity], out_hbm.at[pl.ds(base + c * _OC, _OC)],
                osem.at[parity])

        for g in range(_NBUF):
            pltpu.make_async_copy(
                emb_hbm.at[idx_v.at[pl.ds(g * _GR * _LP, _GR * _LP)]],
                ring_v.at[g], gsem.at[g]).wait()
        for par in range(2):
            pltpu.make_async_copy(
                outb_v.at[par], out_hbm.at[pl.ds(base, _OC)],
                osem.at[par]).wait()

    return sc_pool


def _tc_head(x_all, sums, e0, wts, bs, gs, betas, block_m, nq_blocks):
    btot = sums.shape[0]

    def body(x_ref, s_ref, e0_ref, w_ref, b_ref, g_ref, be_ref, o_ref):
        x = x_ref[...]
        n0 = jnp.sum((x == 0).astype(jnp.float32), axis=1, keepdims=True)
        cnt = jnp.maximum(jnp.float32(_L) - n0, 1.0)
        pooled = (s_ref[...] - n0 * e0_ref[0, :][None, :]) / cnt
        h = jnp.dot(pooled, w_ref[0], preferred_element_type=jnp.float32)
        h = jnp.maximum(h + b_ref[0], 0.0)
        mu = jnp.mean(h, axis=1, keepdims=True)
        var = jnp.mean((h - mu) ** 2, axis=1, keepdims=True)
        hn = (h - mu) * lax.rsqrt(var + 1e-5)
        hn = hn * g_ref[0] + be_ref[0]
        nrm = jnp.sqrt(jnp.sum(hn * hn, axis=1, keepdims=True))
        o_ref[...] = hn / jnp.maximum(nrm, 1e-12)

    def w_idx(i):
        return (jnp.minimum(i // nq_blocks, 1), 0, 0)

    return pl.pallas_call(
        body,
        grid=(btot // block_m,),
        in_specs=[
            pl.BlockSpec((block_m, _L), lambda i: (i, 0)),
            pl.BlockSpec((block_m, _D), lambda i: (i, 0)),
            pl.BlockSpec((1, _D), lambda i: (0, 0)),
            pl.BlockSpec((1, _D, _D),
                         lambda i: (jnp.minimum(i // nq_blocks, 1), 0, 0)),
            pl.BlockSpec((1, 1, _D), w_idx),
            pl.BlockSpec((1, 1, _D), w_idx),
            pl.BlockSpec((1, 1, _D), w_idx),
        ],
        out_specs=pl.BlockSpec((block_m, _D), lambda i: (i, 0)),
        out_shape=jax.ShapeDtypeStruct((btot, _D), jnp.float32),
    )(x_all, sums, e0, wts, bs, gs, betas)


def kernel(q, p, n, emb, Wq, bq, gq, betaq, Wd, bd, gd, betad):
    b = q.shape[0]
    x_all = jnp.concatenate([q, p, n], axis=0).astype(jnp.int32)
    emb = emb.astype(jnp.float32)
    embp = jnp.pad(emb, ((0, 0), (0, _DP - _D)))
    xp = jnp.pad(x_all, ((0, 0), (0, _LP - _L))).reshape(-1)
    sums = _build_sc_pool(x_all.shape[0])(xp, embp)
    e0 = emb[0:1]
    wts = jnp.stack([Wq.T, Wd.T])
    bs = jnp.stack([bq, bd])[:, None, :]
    gs = jnp.stack([gq, gd])[:, None, :]
    betas = jnp.stack([betaq, betad])[:, None, :]
    block_m = 256
    enc = _tc_head(x_all, sums, e0, wts, bs, gs, betas, block_m, b // block_m)
    return enc[:b], enc[b:2 * b], enc[2 * b:]


# speedup vs baseline: 1.0996x; 1.0996x over previous
import functools

import jax
import jax.numpy as jnp
from jax import lax
from jax.experimental import pallas as pl
from jax.experimental.pallas import tpu as pltpu
from jax.experimental.pallas import tpu_sc as plsc

_D = 300
_DP = 384
_L = 50
_LP = 56
_LANES = 16
_GR = 2
_NBUF = 2
_OC = 16

_OFFS = tuple(list(range(0, 288, _LANES)) + [_D - _LANES])


def _build_sc_pool(btot):
    info = plsc.get_sparse_core_info()
    nc, ns = info.num_cores, info.num_subcores
    nw = nc * ns
    rpw = btot // nw
    assert btot % nw == 0 and rpw % _OC == 0
    nch = rpw // _OC
    mesh = plsc.VectorSubcoreMesh(core_axis_name="c", subcore_axis_name="s")

    @functools.partial(
        pl.kernel,
        out_type=jax.ShapeDtypeStruct((btot, _D), jnp.float32),
        mesh=mesh,
        scratch_types=[
            pltpu.VMEM((rpw * _LP,), jnp.int32),
            pltpu.VMEM((_NBUF, _GR * _LP, _DP), jnp.float32),
            pltpu.VMEM((2, _OC, _D), jnp.float32),
            pltpu.SemaphoreType.DMA((_NBUF,)),
            pltpu.SemaphoreType.DMA((2,)),
        ],
    )
    def sc_pool(x_hbm, emb_hbm, out_hbm, idx_v, ring_v, outb_v, gsem, osem):
        wid = lax.axis_index("s") * nc + lax.axis_index("c")
        base = wid * rpw
        pltpu.sync_copy(x_hbm.at[pl.ds(base * _LP, rpw * _LP)], idx_v)

        for g in range(_NBUF):
            pltpu.async_copy(
                emb_hbm.at[idx_v.at[pl.ds(g * _GR * _LP, _GR * _LP)]],
                ring_v.at[g], gsem.at[g])

        @pl.loop(0, nch)
        def _chunk(c):
            parity = lax.rem(c, 2)

            @pl.when(c >= 2)
            def _():
                pltpu.make_async_copy(
                    outb_v.at[parity], out_hbm.at[pl.ds(base, _OC)],
                    osem.at[parity]).wait()

            for j in range(0, _OC, _GR):
                row = c * _OC + j
                g = j // _GR
                slot = g % _NBUF
                pltpu.make_async_copy(
                    emb_hbm.at[idx_v.at[pl.ds(row * _LP, _GR * _LP)]],
                    ring_v.at[slot], gsem.at[slot]).wait()
                rv = ring_v.at[slot]
                for s in range(_GR):
                    acc0 = tuple(
                        rv[s * _LP, pl.ds(o, _LANES)] for o in _OFFS)

                    def _body(r, acc, rv=rv, s=s):
                        return tuple(
                            a + rv[s * _LP + r, pl.ds(o, _LANES)]
                            for a, o in zip(acc, _OFFS))

                    acc = lax.fori_loop(1, _L, _body, acc0)
                    for t, o in enumerate(_OFFS):
                        outb_v[parity, j + s, pl.ds(o, _LANES)] = acc[t]
                nxt = jnp.minimum(row + _NBUF * _GR, rpw - _GR)
                pltpu.async_copy(
                    emb_hbm.at[idx_v.at[pl.ds(nxt * _LP, _GR * _LP)]],
                    ring_v.at[slot], gsem.at[slot])

            pltpu.async_copy(
                outb_v.at[parity], out_hbm.at[pl.ds(base + c * _OC, _OC)],
                osem.at[parity])

        for g in range(_NBUF):
            pltpu.make_async_copy(
                emb_hbm.at[idx_v.at[pl.ds(g * _GR * _LP, _GR * _LP)]],
                ring_v.at[g], gsem.at[g]).wait()
        for par in range(2):
            pltpu.make_async_copy(
                outb_v.at[par], out_hbm.at[pl.ds(base, _OC)],
                osem.at[par]).wait()

    return sc_pool


def _tc_head(x_all, sums, e0, wts, bs, gs, betas, block_m, nq_blocks):
    btot = sums.shape[0]

    def body(x_ref, s_ref, e0_ref, w_ref, b_ref, g_ref, be_ref, o_ref):
        x = x_ref[...]
        n0 = jnp.sum((x == 0).astype(jnp.float32), axis=1, keepdims=True)
        cnt = jnp.maximum(jnp.float32(_L) - n0, 1.0)
        pooled = (s_ref[...] - n0 * e0_ref[0, :][None, :]) / cnt
        h = jnp.dot(pooled, w_ref[0], preferred_element_type=jnp.float32)
        h = jnp.maximum(h + b_ref[0], 0.0)
        mu = jnp.mean(h, axis=1, keepdims=True)
        var = jnp.mean((h - mu) ** 2, axis=1, keepdims=True)
        hn = (h - mu) * lax.rsqrt(var + 1e-5)
        hn = hn * g_ref[0] + be_ref[0]
        nrm = jnp.sqrt(jnp.sum(hn * hn, axis=1, keepdims=True))
        o_ref[...] = hn / jnp.maximum(nrm, 1e-12)

    def w_idx(i):
        return (jnp.minimum(i // nq_blocks, 1), 0, 0)

    return pl.pallas_call(
        body,
        grid=(btot // block_m,),
        in_specs=[
            pl.BlockSpec((block_m, _L), lambda i: (i, 0)),
            pl.BlockSpec((block_m, _D), lambda i: (i, 0)),
            pl.BlockSpec((1, _D), lambda i: (0, 0)),
            pl.BlockSpec((1, _D, _D),
                         lambda i: (jnp.minimum(i // nq_blocks, 1), 0, 0)),
            pl.BlockSpec((1, 1, _D), w_idx),
            pl.BlockSpec((1, 1, _D), w_idx),
            pl.BlockSpec((1, 1, _D), w_idx),
        ],
        out_specs=pl.BlockSpec((block_m, _D), lambda i: (i, 0)),
        out_shape=jax.ShapeDtypeStruct((btot, _D), jnp.float32),
    )(x_all, sums, e0, wts, bs, gs, betas)


def _tc_pad(emb):
    v = emb.shape[0]
    br = 2000

    def body(x_ref, o_ref):
        o_ref[:, :_D] = x_ref[...]
        o_ref[:, _D:] = jnp.zeros((br, _DP - _D), jnp.float32)

    return pl.pallas_call(
        body,
        grid=(v // br,),
        in_specs=[pl.BlockSpec((br, _D), lambda i: (i, 0))],
        out_specs=pl.BlockSpec((br, _DP), lambda i: (i, 0)),
        out_shape=jax.ShapeDtypeStruct((v, _DP), jnp.float32),
    )(emb)


def kernel(q, p, n, emb, Wq, bq, gq, betaq, Wd, bd, gd, betad):
    b = q.shape[0]
    x_all = jnp.concatenate([q, p, n], axis=0).astype(jnp.int32)
    emb = emb.astype(jnp.float32)
    embp = _tc_pad(emb)
    xp = jnp.pad(x_all, ((0, 0), (0, _LP - _L))).reshape(-1)
    sums = _build_sc_pool(x_all.shape[0])(xp, embp)
    e0 = emb[0:1]
    wts = jnp.stack([Wq.T, Wd.T])
    bs = jnp.stack([bq, bd])[:, None, :]
    gs = jnp.stack([gq, gd])[:, None, :]
    betas = jnp.stack([betaq, betad])[:, None, :]
    block_m = 256
    enc = _tc_head(x_all, sums, e0, wts, bs, gs, betas, block_m, b // block_m)
    return enc[:b], enc[b:2 * b], enc[2 * b:]


# speedup vs baseline: 2.3192x; 2.1092x over previous
import functools

import jax
import jax.numpy as jnp
from jax import lax
from jax.experimental import pallas as pl
from jax.experimental.pallas import tpu as pltpu
from jax.experimental.pallas import tpu_sc as plsc

_D = 300
_DP = 384
_L = 50
_PR = 2 * _L + 4
_LANES = 16
_NBUF = 2
_OC = 16

_OFFS = tuple(list(range(0, 288, _LANES)) + [_D - _LANES])


def _build_sc_pool(btot):
    info = plsc.get_sparse_core_info()
    nc, ns = info.num_cores, info.num_subcores
    nw = nc * ns
    rpw = btot // nw
    npairs = rpw // 2
    assert btot % nw == 0 and rpw % _OC == 0
    nch = rpw // _OC
    mesh = plsc.VectorSubcoreMesh(core_axis_name="c", subcore_axis_name="s")

    @functools.partial(
        pl.kernel,
        out_type=jax.ShapeDtypeStruct((btot, _D), jnp.float32),
        mesh=mesh,
        scratch_types=[
            pltpu.VMEM((npairs * _PR,), jnp.int32),
            pltpu.VMEM((_NBUF, _PR, _DP), jnp.float32),
            pltpu.VMEM((2, _OC, _D), jnp.float32),
            pltpu.SemaphoreType.DMA((_NBUF,)),
            pltpu.SemaphoreType.DMA((2,)),
        ],
    )
    def sc_pool(x_hbm, emb_hbm, out_hbm, idx_v, ring_v, outb_v, gsem, osem):
        wid = lax.axis_index("s") * nc + lax.axis_index("c")
        base = wid * rpw
        pltpu.sync_copy(
            x_hbm.at[pl.ds(wid * npairs * _PR, npairs * _PR)], idx_v)

        for g in range(_NBUF):
            pltpu.async_copy(
                emb_hbm.at[idx_v.at[pl.ds(g * _PR, _PR)]],
                ring_v.at[g], gsem.at[g])

        @pl.loop(0, nch)
        def _chunk(c):
            parity = lax.rem(c, 2)

            @pl.when(c >= 2)
            def _():
                pltpu.make_async_copy(
                    outb_v.at[parity], out_hbm.at[pl.ds(base, _OC)],
                    osem.at[parity]).wait()

            for j in range(0, _OC, 2):
                pair = c * (_OC // 2) + j // 2
                slot = (j // 2) % _NBUF
                pltpu.make_async_copy(
                    emb_hbm.at[idx_v.at[pl.ds(pair * _PR, _PR)]],
                    ring_v.at[slot], gsem.at[slot]).wait()
                rv = ring_v.at[slot]
                for s in range(2):
                    acc0 = tuple(
                        rv[s * _L, pl.ds(o, _LANES)] for o in _OFFS)

                    def _body(r, acc, rv=rv, s=s):
                        return tuple(
                            a + rv[s * _L + r, pl.ds(o, _LANES)]
                            for a, o in zip(acc, _OFFS))

                    acc = lax.fori_loop(1, _L, _body, acc0)
                    for t, o in enumerate(_OFFS):
                        outb_v[parity, j + s, pl.ds(o, _LANES)] = acc[t]
                nxt = jnp.minimum(pair + _NBUF, npairs - 1)
                pltpu.async_copy(
                    emb_hbm.at[idx_v.at[pl.ds(nxt * _PR, _PR)]],
                    ring_v.at[slot], gsem.at[slot])

            pltpu.async_copy(
                outb_v.at[parity], out_hbm.at[pl.ds(base + c * _OC, _OC)],
                osem.at[parity])

        for g in range(_NBUF):
            pltpu.make_async_copy(
                emb_hbm.at[idx_v.at[pl.ds(g * _PR, _PR)]],
                ring_v.at[g], gsem.at[g]).wait()
        for par in range(2):
            pltpu.make_async_copy(
                outb_v.at[par], out_hbm.at[pl.ds(base, _OC)],
                osem.at[par]).wait()

    return sc_pool


def _tc_head(x_all, sums, e0, wts, bs, gs, betas, block_m, nq_blocks):
    btot = sums.shape[0]

    def body(x_ref, s_ref, e0_ref, w_ref, b_ref, g_ref, be_ref, o_ref):
        x = x_ref[...]
        n0 = jnp.sum((x == 0).astype(jnp.float32), axis=1, keepdims=True)
        cnt = jnp.maximum(jnp.float32(_L) - n0, 1.0)
        pooled = (s_ref[...] - n0 * e0_ref[0, :][None, :]) / cnt
        h = jnp.dot(pooled, w_ref[0], preferred_element_type=jnp.float32)
        h = jnp.maximum(h + b_ref[0], 0.0)
        mu = jnp.mean(h, axis=1, keepdims=True)
        var = jnp.mean((h - mu) ** 2, axis=1, keepdims=True)
        hn = (h - mu) * lax.rsqrt(var + 1e-5)
        hn = hn * g_ref[0] + be_ref[0]
        nrm = jnp.sqrt(jnp.sum(hn * hn, axis=1, keepdims=True))
        o_ref[...] = hn / jnp.maximum(nrm, 1e-12)

    def w_idx(i):
        return (jnp.minimum(i // nq_blocks, 1), 0, 0)

    return pl.pallas_call(
        body,
        grid=(btot // block_m,),
        in_specs=[
            pl.BlockSpec((block_m, _L), lambda i: (i, 0)),
            pl.BlockSpec((block_m, _D), lambda i: (i, 0)),
            pl.BlockSpec((1, _D), lambda i: (0, 0)),
            pl.BlockSpec((1, _D, _D),
                         lambda i: (jnp.minimum(i // nq_blocks, 1), 0, 0)),
            pl.BlockSpec((1, 1, _D), w_idx),
            pl.BlockSpec((1, 1, _D), w_idx),
            pl.BlockSpec((1, 1, _D), w_idx),
        ],
        out_specs=pl.BlockSpec((block_m, _D), lambda i: (i, 0)),
        out_shape=jax.ShapeDtypeStruct((btot, _D), jnp.float32),
    )(x_all, sums, e0, wts, bs, gs, betas)


def _tc_pad(emb):
    v = emb.shape[0]
    br = 2000

    def body(x_ref, o_ref):
        o_ref[:, :_D] = x_ref[...]
        o_ref[:, _D:] = jnp.zeros((br, _DP - _D), jnp.float32)

    return pl.pallas_call(
        body,
        grid=(v // br,),
        in_specs=[pl.BlockSpec((br, _D), lambda i: (i, 0))],
        out_specs=pl.BlockSpec((br, _DP), lambda i: (i, 0)),
        out_shape=jax.ShapeDtypeStruct((v, _DP), jnp.float32),
    )(emb)


def kernel(q, p, n, emb, Wq, bq, gq, betaq, Wd, bd, gd, betad):
    b = q.shape[0]
    x_all = jnp.concatenate([q, p, n], axis=0).astype(jnp.int32)
    emb = emb.astype(jnp.float32)
    embp = _tc_pad(emb)
    xp = jnp.pad(x_all.reshape(-1, 2 * _L), ((0, 0), (0, _PR - 2 * _L)))
    sums = _build_sc_pool(x_all.shape[0])(xp.reshape(-1), embp)
    e0 = emb[0:1]
    wts = jnp.stack([Wq.T, Wd.T])
    bs = jnp.stack([bq, bd])[:, None, :]
    gs = jnp.stack([gq, gd])[:, None, :]
    betas = jnp.stack([betaq, betad])[:, None, :]
    block_m = 256
    enc = _tc_head(x_all, sums, e0, wts, bs, gs, betas, block_m, b // block_m)
    return enc[:b], enc[b:2 * b], enc[2 * b:]


# speedup vs baseline: 5.4550x; 2.3521x over previous
import functools

import jax
import jax.numpy as jnp
from jax import lax
from jax.experimental import pallas as pl
from jax.experimental.pallas import tpu as pltpu
from jax.experimental.pallas import tpu_sc as plsc

_D = 300
_DP = 384
_L = 50
_PR = 2 * _L + 4
_LANES = 16
_NBUF = 2
_OC = 16

_OFFS = tuple(list(range(0, 288, _LANES)) + [_D - _LANES])


def _build_sc_pool(btot):
    info = plsc.get_sparse_core_info()
    nc, ns = info.num_cores, info.num_subcores
    nw = nc * ns
    rpw = btot // nw
    npairs = rpw // 2
    assert btot % nw == 0 and rpw % _OC == 0
    nch = rpw // _OC
    mesh = plsc.VectorSubcoreMesh(core_axis_name="c", subcore_axis_name="s")

    @functools.partial(
        pl.kernel,
        out_type=jax.ShapeDtypeStruct((btot, _D), jnp.float32),
        mesh=mesh,
        scratch_types=[
            pltpu.VMEM((npairs * _PR,), jnp.int32),
            pltpu.VMEM((_NBUF, _PR, _DP), jnp.float32),
            pltpu.VMEM((2, _OC, _D), jnp.float32),
            pltpu.SemaphoreType.DMA((_NBUF,)),
            pltpu.SemaphoreType.DMA((2,)),
        ],
    )
    def sc_pool(x_hbm, emb_hbm, out_hbm, idx_v, ring_v, outb_v, gsem, osem):
        wid = lax.axis_index("s") * nc + lax.axis_index("c")
        base = wid * rpw
        pltpu.sync_copy(
            x_hbm.at[pl.ds(wid * npairs * _PR, npairs * _PR)], idx_v)

        for g in range(_NBUF):
            pltpu.async_copy(
                emb_hbm.at[idx_v.at[pl.ds(g * _PR, _PR)]],
                ring_v.at[g], gsem.at[g])

        @pl.loop(0, nch)
        def _chunk(c):
            parity = lax.rem(c, 2)

            @pl.when(c >= 2)
            def _():
                pltpu.make_async_copy(
                    outb_v.at[parity], out_hbm.at[pl.ds(base, _OC)],
                    osem.at[parity]).wait()

            for j in range(0, _OC, 2):
                pair = c * (_OC // 2) + j // 2
                slot = (j // 2) % _NBUF
                pltpu.make_async_copy(
                    emb_hbm.at[idx_v.at[pl.ds(pair * _PR, _PR)]],
                    ring_v.at[slot], gsem.at[slot]).wait()
                rv = ring_v.at[slot]
                for s in range(2):
                    acc0 = tuple(
                        rv[s * _L, pl.ds(o, _LANES)] for o in _OFFS)

                    def _body(r, acc, rv=rv, s=s):
                        return tuple(
                            a + rv[s * _L + r, pl.ds(o, _LANES)]
                            for a, o in zip(acc, _OFFS))

                    acc = lax.fori_loop(1, _L, _body, acc0)
                    for t, o in enumerate(_OFFS):
                        outb_v[parity, j + s, pl.ds(o, _LANES)] = acc[t]
                nxt = jnp.minimum(pair + _NBUF, npairs - 1)
                pltpu.async_copy(
                    emb_hbm.at[idx_v.at[pl.ds(nxt * _PR, _PR)]],
                    ring_v.at[slot], gsem.at[slot])

            pltpu.async_copy(
                outb_v.at[parity], out_hbm.at[pl.ds(base + c * _OC, _OC)],
                osem.at[parity])

        for g in range(_NBUF):
            pltpu.make_async_copy(
                emb_hbm.at[idx_v.at[pl.ds(g * _PR, _PR)]],
                ring_v.at[g], gsem.at[g]).wait()
        for par in range(2):
            pltpu.make_async_copy(
                outb_v.at[par], out_hbm.at[pl.ds(base, _OC)],
                osem.at[par]).wait()

    return sc_pool


def _tc_head(x_all, sums, e0, wts, bs, gs, betas, block_m, nq_blocks):
    btot = sums.shape[0]

    def body(x_ref, s_ref, e0_ref, w_ref, b_ref, g_ref, be_ref, o_ref):
        x = x_ref[...]
        n0 = jnp.sum((x == 0).astype(jnp.float32), axis=1, keepdims=True)
        cnt = jnp.maximum(jnp.float32(_L) - n0, 1.0)
        pooled = (s_ref[...] - n0 * e0_ref[0, :][None, :]) / cnt
        h = jnp.dot(pooled, w_ref[0], preferred_element_type=jnp.float32)
        h = jnp.maximum(h + b_ref[0], 0.0)
        mu = jnp.mean(h, axis=1, keepdims=True)
        var = jnp.mean((h - mu) ** 2, axis=1, keepdims=True)
        hn = (h - mu) * lax.rsqrt(var + 1e-5)
        hn = hn * g_ref[0] + be_ref[0]
        nrm = jnp.sqrt(jnp.sum(hn * hn, axis=1, keepdims=True))
        o_ref[...] = hn / jnp.maximum(nrm, 1e-12)

    def w_idx(i):
        return (jnp.minimum(i // nq_blocks, 1), 0, 0)

    return pl.pallas_call(
        body,
        grid=(btot // block_m,),
        in_specs=[
            pl.BlockSpec((block_m, _L), lambda i: (i, 0)),
            pl.BlockSpec((block_m, _D), lambda i: (i, 0)),
            pl.BlockSpec((1, _D), lambda i: (0, 0)),
            pl.BlockSpec((1, _D, _D),
                         lambda i: (jnp.minimum(i // nq_blocks, 1), 0, 0)),
            pl.BlockSpec((1, 1, _D), w_idx),
            pl.BlockSpec((1, 1, _D), w_idx),
            pl.BlockSpec((1, 1, _D), w_idx),
        ],
        out_specs=pl.BlockSpec((block_m, _D), lambda i: (i, 0)),
        out_shape=jax.ShapeDtypeStruct((btot, _D), jnp.float32),
    )(x_all, sums, e0, wts, bs, gs, betas)


def _tc_pad(emb):
    v = emb.shape[0]
    br = 2000

    def body(x_ref, o_ref):
        o_ref[:, :_D] = x_ref[...]
        o_ref[:, _D:] = jnp.zeros((br, _DP - _D), jnp.float32)

    return pl.pallas_call(
        body,
        grid=(v // br,),
        in_specs=[pl.BlockSpec((br, _D), lambda i: (i, 0))],
        out_specs=pl.BlockSpec((br, _DP), lambda i: (i, 0)),
        out_shape=jax.ShapeDtypeStruct((v, _DP), jnp.float32),
    )(emb)


def kernel(q, p, n, emb, Wq, bq, gq, betaq, Wd, bd, gd, betad):
    b = q.shape[0]
    x_all = jnp.concatenate([q, p, n], axis=0).astype(jnp.int32)
    emb = emb.astype(jnp.float32)
    embp = _tc_pad(emb)
    xpairs = x_all.reshape(-1, 2 * _L)
    fill = (jnp.arange(xpairs.shape[0], dtype=jnp.int32)[:, None] * 4
            + jnp.arange(_PR - 2 * _L, dtype=jnp.int32)[None, :]) % emb.shape[0]
    xp = jnp.concatenate([xpairs, fill], axis=1)
    sums = _build_sc_pool(x_all.shape[0])(xp.reshape(-1), embp)
    e0 = emb[0:1]
    wts = jnp.stack([Wq.T, Wd.T])
    bs = jnp.stack([bq, bd])[:, None, :]
    gs = jnp.stack([gq, gd])[:, None, :]
    betas = jnp.stack([betaq, betad])[:, None, :]
    block_m = 256
    enc = _tc_head(x_all, sums, e0, wts, bs, gs, betas, block_m, b // block_m)
    return enc[:b], enc[b:2 * b], enc[2 * b:]


# speedup vs baseline: 5.7116x; 1.0470x over previous
import functools

import jax
import jax.numpy as jnp
from jax import lax
from jax.experimental import pallas as pl
from jax.experimental.pallas import tpu as pltpu
from jax.experimental.pallas import tpu_sc as plsc

_D = 300
_DP = 384
_L = 50
_PR = _L + 6
_LANES = 16
_NBUF = 4
_OC = 16

_OFFS = tuple(list(range(0, 288, _LANES)) + [_D - _LANES])


def _build_sc_pool(btot):
    info = plsc.get_sparse_core_info()
    nc, ns = info.num_cores, info.num_subcores
    nw = nc * ns
    rpw = btot // nw
    npairs = rpw
    assert btot % nw == 0 and rpw % _OC == 0
    nch = rpw // _OC
    mesh = plsc.VectorSubcoreMesh(core_axis_name="c", subcore_axis_name="s")

    @functools.partial(
        pl.kernel,
        out_type=jax.ShapeDtypeStruct((btot, _D), jnp.float32),
        mesh=mesh,
        scratch_types=[
            pltpu.VMEM((npairs * _PR,), jnp.int32),
            pltpu.VMEM((_NBUF, _PR, _DP), jnp.float32),
            pltpu.VMEM((2, _OC, _D), jnp.float32),
            pltpu.SemaphoreType.DMA((_NBUF,)),
            pltpu.SemaphoreType.DMA((2,)),
        ],
    )
    def sc_pool(x_hbm, emb_hbm, out_hbm, idx_v, ring_v, outb_v, gsem, osem):
        wid = lax.axis_index("s") * nc + lax.axis_index("c")
        base = wid * rpw
        pltpu.sync_copy(
            x_hbm.at[pl.ds(wid * npairs * _PR, npairs * _PR)], idx_v)

        for g in range(_NBUF):
            pltpu.async_copy(
                emb_hbm.at[idx_v.at[pl.ds(g * _PR, _PR)]],
                ring_v.at[g], gsem.at[g])

        @pl.loop(0, nch)
        def _chunk(c):
            parity = lax.rem(c, 2)

            @pl.when(c >= 2)
            def _():
                pltpu.make_async_copy(
                    outb_v.at[parity], out_hbm.at[pl.ds(base, _OC)],
                    osem.at[parity]).wait()

            for j in range(_OC):
                pair = c * _OC + j
                slot = j % _NBUF
                pltpu.make_async_copy(
                    emb_hbm.at[idx_v.at[pl.ds(pair * _PR, _PR)]],
                    ring_v.at[slot], gsem.at[slot]).wait()
                rv = ring_v.at[slot]
                acc0 = tuple(rv[0, pl.ds(o, _LANES)] for o in _OFFS)

                def _body(r, acc, rv=rv):
                    return tuple(
                        a + rv[r, pl.ds(o, _LANES)]
                        for a, o in zip(acc, _OFFS))

                acc = lax.fori_loop(1, _L, _body, acc0)
                for t, o in enumerate(_OFFS):
                    outb_v[parity, j, pl.ds(o, _LANES)] = acc[t]
                nxt = jnp.minimum(pair + _NBUF, npairs - 1)
                pltpu.async_copy(
                    emb_hbm.at[idx_v.at[pl.ds(nxt * _PR, _PR)]],
                    ring_v.at[slot], gsem.at[slot])

            pltpu.async_copy(
                outb_v.at[parity], out_hbm.at[pl.ds(base + c * _OC, _OC)],
                osem.at[parity])

        for g in range(_NBUF):
            pltpu.make_async_copy(
                emb_hbm.at[idx_v.at[pl.ds(g * _PR, _PR)]],
                ring_v.at[g], gsem.at[g]).wait()
        for par in range(2):
            pltpu.make_async_copy(
                outb_v.at[par], out_hbm.at[pl.ds(base, _OC)],
                osem.at[par]).wait()

    return sc_pool


def _tc_head(x_all, sums, e0, wts, bs, gs, betas, block_m, nq_blocks):
    btot = sums.shape[0]

    def body(x_ref, s_ref, e0_ref, w_ref, b_ref, g_ref, be_ref, o_ref):
        x = x_ref[...]
        n0 = jnp.sum((x == 0).astype(jnp.float32), axis=1, keepdims=True)
        cnt = jnp.maximum(jnp.float32(_L) - n0, 1.0)
        pooled = (s_ref[...] - n0 * e0_ref[0, :][None, :]) / cnt
        h = jnp.dot(pooled, w_ref[0], preferred_element_type=jnp.float32)
        h = jnp.maximum(h + b_ref[0], 0.0)
        mu = jnp.mean(h, axis=1, keepdims=True)
        var = jnp.mean((h - mu) ** 2, axis=1, keepdims=True)
        hn = (h - mu) * lax.rsqrt(var + 1e-5)
        hn = hn * g_ref[0] + be_ref[0]
        nrm = jnp.sqrt(jnp.sum(hn * hn, axis=1, keepdims=True))
        o_ref[...] = hn / jnp.maximum(nrm, 1e-12)

    def w_idx(i):
        return (jnp.minimum(i // nq_blocks, 1), 0, 0)

    return pl.pallas_call(
        body,
        grid=(btot // block_m,),
        in_specs=[
            pl.BlockSpec((block_m, _L), lambda i: (i, 0)),
            pl.BlockSpec((block_m, _D), lambda i: (i, 0)),
            pl.BlockSpec((1, _D), lambda i: (0, 0)),
            pl.BlockSpec((1, _D, _D),
                         lambda i: (jnp.minimum(i // nq_blocks, 1), 0, 0)),
            pl.BlockSpec((1, 1, _D), w_idx),
            pl.BlockSpec((1, 1, _D), w_idx),
            pl.BlockSpec((1, 1, _D), w_idx),
        ],
        out_specs=pl.BlockSpec((block_m, _D), lambda i: (i, 0)),
        out_shape=jax.ShapeDtypeStruct((btot, _D), jnp.float32),
    )(x_all, sums, e0, wts, bs, gs, betas)


def _tc_pad(emb):
    v = emb.shape[0]
    br = 2000

    def body(x_ref, o_ref):
        o_ref[:, :_D] = x_ref[...]
        o_ref[:, _D:] = jnp.zeros((br, _DP - _D), jnp.float32)

    return pl.pallas_call(
        body,
        grid=(v // br,),
        in_specs=[pl.BlockSpec((br, _D), lambda i: (i, 0))],
        out_specs=pl.BlockSpec((br, _DP), lambda i: (i, 0)),
        out_shape=jax.ShapeDtypeStruct((v, _DP), jnp.float32),
    )(emb)


def kernel(q, p, n, emb, Wq, bq, gq, betaq, Wd, bd, gd, betad):
    b = q.shape[0]
    x_all = jnp.concatenate([q, p, n], axis=0).astype(jnp.int32)
    emb = emb.astype(jnp.float32)
    embp = _tc_pad(emb)
    xrows = x_all
    fill = (jnp.arange(xrows.shape[0], dtype=jnp.int32)[:, None] * 8
            + jnp.arange(_PR - _L, dtype=jnp.int32)[None, :]) % emb.shape[0]
    xp = jnp.concatenate([xrows, fill], axis=1)
    sums = _build_sc_pool(x_all.shape[0])(xp.reshape(-1), embp)
    e0 = emb[0:1]
    wts = jnp.stack([Wq.T, Wd.T])
    bs = jnp.stack([bq, bd])[:, None, :]
    gs = jnp.stack([gq, gd])[:, None, :]
    betas = jnp.stack([betaq, betad])[:, None, :]
    block_m = 256
    enc = _tc_head(x_all, sums, e0, wts, bs, gs, betas, block_m, b // block_m)
    return enc[:b], enc[b:2 * b], enc[2 * b:]
